# baseline (speedup 1.0000x reference)
import functools

import jax
import jax.numpy as jnp
from jax import lax
from jax.experimental import pallas as pl
from jax.experimental.pallas import tpu as pltpu
from jax.experimental.pallas import tpu_sc as plsc

NC = 2
NS = 16
L = 16
NW = NC * NS


def _head_body(x_ref, w_ref, asr_ref, adr_ref, h_ref, as_ref, ad_ref,
               m_ref, ws_ref):
    h = jnp.dot(x_ref[...], w_ref[...], preferred_element_type=jnp.float32)
    h_ref[...] = h
    as_ = jnp.sum(h * asr_ref[...], axis=1, keepdims=True)
    ad_ = jnp.sum(h * adr_ref[...], axis=1, keepdims=True)
    as_ref[...] = as_
    ad_ref[...] = ad_
    m = jnp.max(as_) + jnp.max(ad_)
    m = jnp.where(m >= 0.0, m, 0.2 * m)
    m_ref[...] = jnp.full((1, L), m, jnp.float32)
    y = as_ + ad_
    z = jnp.maximum(y, 0.2 * y)
    ws_ref[...] = jnp.exp(z - m)


def _head(x, W, a_src, a_dst):
    N, D = x.shape
    f32 = jnp.float32
    return pl.pallas_call(
        _head_body,
        out_shape=[
            jax.ShapeDtypeStruct((N, D), f32),
            jax.ShapeDtypeStruct((N, 1), f32),
            jax.ShapeDtypeStruct((N, 1), f32),
            jax.ShapeDtypeStruct((1, L), f32),
            jax.ShapeDtypeStruct((N, 1), f32),
        ],
    )(x, W, a_src.reshape(1, D), a_dst.reshape(1, D))


def _combine_body(acc_ref, dent_ref, h_ref, ws_ref, b_ref, o_ref, *, act, D):
    ws = ws_ref[...]
    N = ws_ref.shape[0]
    num = acc_ref[0, :N, :] + acc_ref[1, :N, :] + ws * h_ref[...]
    den = jnp.sum(dent_ref[...], axis=1, keepdims=True) + ws
    out = num / (den + 1e-16) + b_ref[...]
    if act == "relu":
        out = jnp.maximum(out, 0.0)
    else:
        mx = jnp.max(out, axis=1, keepdims=True)
        sh = out - mx
        out = sh - jnp.log(jnp.sum(jnp.exp(sh), axis=1, keepdims=True))
    o_ref[...] = out


def _combine(acc, den_parts, h, ws, b, act):
    N, D = h.shape
    den_t = den_parts.T
    return pl.pallas_call(
        functools.partial(_combine_body, act=act, D=D),
        out_shape=jax.ShapeDtypeStruct((N, D), jnp.float32),
    )(acc, den_t, h, ws, b.reshape(1, D))


def _mid_body(acc_ref, dent_ref, h_ref, ws_ref, b_ref, w2_ref, asr2_ref,
              adr2_ref, h2_ref, as2_ref, ad2_ref, m2_ref, ws2_ref, *, D):
    ws = ws_ref[...]
    N = ws_ref.shape[0]
    num = acc_ref[0, :N, :] + acc_ref[1, :N, :] + ws * h_ref[...]
    den = jnp.sum(dent_ref[...], axis=1, keepdims=True) + ws
    o1 = num / (den + 1e-16) + b_ref[...]
    o1 = jnp.maximum(o1, 0.0)
    h2 = jnp.dot(o1, w2_ref[...], preferred_element_type=jnp.float32)
    h2_ref[...] = h2
    as2 = jnp.sum(h2 * asr2_ref[...], axis=1, keepdims=True)
    ad2 = jnp.sum(h2 * adr2_ref[...], axis=1, keepdims=True)
    as2_ref[...] = as2
    ad2_ref[...] = ad2
    m2 = jnp.max(as2) + jnp.max(ad2)
    m2 = jnp.where(m2 >= 0.0, m2, 0.2 * m2)
    m2_ref[...] = jnp.full((1, L), m2, jnp.float32)
    y2 = as2 + ad2
    z2 = jnp.maximum(y2, 0.2 * y2)
    ws2_ref[...] = jnp.exp(z2 - m2)


def _mid(acc, den_parts, h, ws, b, W2, a_src2, a_dst2):
    N, D = h.shape
    den_t = den_parts.T
    f32 = jnp.float32
    return pl.pallas_call(
        functools.partial(_mid_body, D=D),
        out_shape=[
            jax.ShapeDtypeStruct((N, D), f32),
            jax.ShapeDtypeStruct((N, 1), f32),
            jax.ShapeDtypeStruct((N, 1), f32),
            jax.ShapeDtypeStruct((1, L), f32),
            jax.ShapeDtypeStruct((N, 1), f32),
        ],
    )(acc, den_t, h, ws, b.reshape(1, D), W2, a_src2.reshape(1, D),
      a_dst2.reshape(1, D))


NBUF = 3


def _sc_body(src_hbm, dst_hbm, as_hbm, ad_hbm, m_hbm, h_hbm, z_hbm, zn_hbm,
             acc_out, den_out, m_v, den_v, *rings, N, NP, D, E, C):
    RPT = NP // NS
    EPW = E // NW
    NCHUNK = EPW // C
    av_r = rings[0:NBUF]
    dv_r = rings[NBUF:2 * NBUF]
    w_r = rings[2 * NBUF:3 * NBUF]
    dscat_r = rings[3 * NBUF:4 * NBUF]
    grows = rings[4 * NBUF:5 * NBUF]
    sidx_s = rings[5 * NBUF]
    didx_s = rings[5 * NBUF + 1]
    acc_sh = rings[5 * NBUF + 2]
    sem_sf = rings[5 * NBUF + 3]
    sem_g = rings[5 * NBUF + 4:6 * NBUF + 4]
    sem_s = rings[6 * NBUF + 4:7 * NBUF + 4]

    SP = 5
    SLOTW = SP * C

    c = lax.axis_index("c")
    s = lax.axis_index("s")
    wid = s * NC + c
    base = wid * EPW

    pltpu.sync_copy(m_hbm, m_v)
    pltpu.sync_copy(zn_hbm, den_v)
    pltpu.sync_copy(z_hbm, acc_sh.at[pl.ds(s * RPT, RPT)])
    plsc.subcore_barrier()
    mvec = m_v[...]

    def idxoff(gg):
        return ((gg // SP) % 2) * SLOTW + (gg % SP) * C

    def super_desc(sb):
        off = base + sb * SLOTW
        slot = (sb % 2) * SLOTW
        return (
            pltpu.make_async_copy(src_hbm.at[pl.ds(off, SLOTW)],
                                  sidx_s.at[pl.ds(slot, SLOTW)], sem_sf),
            pltpu.make_async_copy(dst_hbm.at[pl.ds(off, SLOTW)],
                                  didx_s.at[pl.ds(slot, SLOTW)], sem_sf),
        )

    def issue_gather(gg, b):
        o1 = idxoff(gg)
        pltpu.async_copy(h_hbm.at[sidx_s.at[pl.ds(o1, C)]], grows[b],
                         sem_g[b])
        pltpu.async_copy(as_hbm.at[sidx_s.at[pl.ds(o1, C)]], av_r[b],
                         sem_g[b])
        pltpu.async_copy(ad_hbm.at[didx_s.at[pl.ds(o1, C)]], dv_r[b],
                         sem_g[b])

    def wait_gather(gg, b):
        o1 = idxoff(gg)
        pltpu.make_async_copy(h_hbm.at[sidx_s.at[pl.ds(o1, C)]], grows[b],
                              sem_g[b]).wait()
        pltpu.make_async_copy(as_hbm.at[sidx_s.at[pl.ds(o1, C)]], av_r[b],
                              sem_g[b]).wait()
        pltpu.make_async_copy(ad_hbm.at[didx_s.at[pl.ds(o1, C)]], dv_r[b],
                              sem_g[b]).wait()

    def scatter_desc(b):
        return pltpu.make_async_copy(grows[b], acc_sh.at[dscat_r[b]],
                                     sem_s[b])

    for d in super_desc(0):
        d.start()
    for d in super_desc(0):
        d.wait()
    issue_gather(0, 0)

    def outer(o, carry):
        for b in range(NBUF):
            g = o * NBUF + b
            b1 = (b + 1) % NBUF

            @pl.when((g % SP == 0) & (g + SP < NCHUNK))
            def _():
                for d in super_desc(g // SP + 1):
                    d.start()

            @pl.when((g % SP == SP - 1) & (g + 1 < NCHUNK))
            def _():
                for d in super_desc((g + 1) // SP):
                    d.wait()

            @pl.when((g >= 2) & (g + 1 < NCHUNK))
            def _():
                scatter_desc(b1).wait()

            @pl.when(g + 1 < NCHUNK)
            def _():
                issue_gather(g + 1, b1)

            @pl.when(g < NCHUNK)
            def _():
                wait_gather(g, b)
                og = idxoff(g)
                for k in range(C // L):
                    a = av_r[b][pl.ds(k * L, L)]
                    d = dv_r[b][pl.ds(k * L, L)]
                    di = didx_s[pl.ds(og + k * L, L)]
                    y = a + d
                    z = jnp.maximum(y, 0.2 * y)
                    w = jnp.exp(z - mvec)
                    w_r[b][pl.ds(k * L, L)] = w
                    dscat_r[b][pl.ds(k * L, L)] = di
                    plsc.addupdate_scatter(den_v, [di], w)

                def srow(j, carry2, b=b):
                    wj = plsc.load_gather(w_r[b],
                                          [jnp.full((L,), j, jnp.int32)])
                    for k in range(D // L):
                        grows[b][j, pl.ds(k * L, L)] = (
                            grows[b][j, pl.ds(k * L, L)] * wj)
                    return carry2

                lax.fori_loop(0, C, srow, 0, unroll=4)
                scatter_desc(b).start(add=True)
        return carry

    NOUTER = (NCHUNK + NBUF - 1) // NBUF
    lax.fori_loop(0, NOUTER, outer, 0)
    for g in (NCHUNK - 3, NCHUNK - 2, NCHUNK - 1):
        scatter_desc(g % NBUF).wait()
    pltpu.sync_copy(den_v, den_out.at[wid])
    plsc.subcore_barrier()
    pltpu.sync_copy(acc_sh.at[pl.ds(s * RPT, RPT)],
                    acc_out.at[c, pl.ds(s * RPT, RPT)])


def _sc_edges(edge_index, as_, ad_, m, h, zeros_nd, zeros_n):
    N, D = h.shape
    E = edge_index.shape[1]
    C = 80
    NP = ((N + NS * 8 - 1) // (NS * 8)) * NS * 8
    assert E % (NW * C) == 0
    f32 = jnp.float32
    mesh = plsc.VectorSubcoreMesh(core_axis_name="c", subcore_axis_name="s",
                                  num_cores=NC, num_subcores=NS)
    body = functools.partial(_sc_body, N=N, NP=NP, D=D, E=E, C=C)
    fn = pl.kernel(
        body,
        out_type=[
            jax.ShapeDtypeStruct((NC, NP, D), f32),
            jax.ShapeDtypeStruct((NW, N), f32),
        ],
        mesh=mesh,
        compiler_params=pltpu.CompilerParams(needs_layout_passes=False),
        scratch_types=(
            [
                pltpu.VMEM((L,), f32),
                pltpu.VMEM((N,), f32),
            ]
            + [pltpu.VMEM((C,), f32) for _ in range(NBUF)]
            + [pltpu.VMEM((C,), f32) for _ in range(NBUF)]
            + [pltpu.VMEM((C,), f32) for _ in range(NBUF)]
            + [pltpu.VMEM((C,), jnp.int32) for _ in range(NBUF)]
            + [pltpu.VMEM((C, D), f32) for _ in range(NBUF)]
            + [pltpu.VMEM((10 * C,), jnp.int32)]
            + [pltpu.VMEM((10 * C,), jnp.int32)]
            + [pltpu.VMEM_SHARED((NP, D), f32)]
            + [pltpu.SemaphoreType.DMA]
            + [pltpu.SemaphoreType.DMA for _ in range(2 * NBUF)]
        ),
    )
    return fn(edge_index[0], edge_index[1], as_.reshape(N), ad_.reshape(N),
              m.reshape(L), h, zeros_nd, zeros_n)


def kernel(x, edge_index, W1, a_src1, a_dst1, b1, W2, a_src2, a_dst2, b2):
    N, D = x.shape
    NP = ((N + NS * 8 - 1) // (NS * 8)) * NS * 8
    zeros_nd = jnp.zeros((NP // NS, D), jnp.float32)
    zeros_n = jnp.zeros((N,), jnp.float32)
    h1, as1, ad1, m1, ws1 = _head(x, W1, a_src1, a_dst1)
    acc1, den1 = _sc_edges(edge_index, as1, ad1, m1, h1, zeros_nd, zeros_n)
    h2, as2, ad2, m2, ws2 = _mid(acc1, den1, h1, ws1, b1, W2, a_src2,
                                 a_dst2)
    acc2, den2 = _sc_edges(edge_index, as2, ad2, m2, h2, zeros_nd, zeros_n)
    return _combine(acc2, den2, h2, ws2, b2, "logsoftmax")
